# baseline (speedup 1.0000x reference)
import functools

import jax
import jax.numpy as jnp
from jax import lax
from jax.experimental import pallas as pl
from jax.experimental.pallas import tpu as pltpu
from jax.experimental.pallas import tpu_sc as plsc

T, D, H, E, K = 2048, 768, 512, 16, 2
TK = T * K
BLK = 128
G = TK // BLK + E
TKP = G * BLK

_NC = 2
_NS = 16
_NW = _NC * _NS
_APW = TK // _NW
_HPW = _APW // 2
_TPW = T // _NW
_LN = 16
_VPR = D // _LN


def _meta_body(e_ref, pos_ref, emap_ref, xmap_ref, used_ref):
    ev = e_ref[...]
    lane_e = lax.broadcasted_iota(jnp.int32, (E, TK), 0)
    oneh = (jnp.broadcast_to(ev, (E, TK)) == lane_e).astype(jnp.int32)

    a = oneh
    k = 1
    while k < TK:
        a = a + jnp.concatenate(
            [jnp.zeros((E, k), jnp.int32), a[:, : TK - k]], axis=1)
        k *= 2
    counts = a[:, TK - 1:TK]
    rank = jnp.sum(oneh * a, axis=0, keepdims=True) - 1

    nblk = (counts + (BLK - 1)) // BLK
    b = nblk
    k = 1
    while k < E:
        b = b + jnp.concatenate(
            [jnp.zeros((k, 1), jnp.int32), b[: E - k, :]], axis=0)
        k *= 2
    blk_cum = b
    bstart = (blk_cum - nblk) * BLK
    pos_ref[...] = (
        jnp.sum(oneh * jnp.broadcast_to(bstart, (E, TK)), axis=0,
                keepdims=True) + rank)

    bid = lax.broadcasted_iota(jnp.int32, (1, G), 1)
    nused = jnp.sum(nblk)
    used = (bid < nused).astype(jnp.int32)
    eb = jnp.sum(
        (jnp.broadcast_to(bid, (E, G)) >= jnp.broadcast_to(blk_cum, (E, G)))
        .astype(jnp.int32), axis=0, keepdims=True)
    eb = jnp.minimum(eb, E - 1)
    last_used = jnp.maximum(nused - 1, 0)
    sel = (bid == last_used).astype(jnp.int32)
    e_last = jnp.sum(sel * eb)
    emap_ref[...] = jnp.where(used == 1, eb, e_last)
    xmap_ref[...] = jnp.minimum(bid, last_used)
    used_ref[...] = used


def _metadata(flat_e):
    pos, emap, xmap, used = pl.pallas_call(
        _meta_body,
        out_shape=(
            jax.ShapeDtypeStruct((1, TK), jnp.int32),
            jax.ShapeDtypeStruct((1, G), jnp.int32),
            jax.ShapeDtypeStruct((1, G), jnp.int32),
            jax.ShapeDtypeStruct((1, G), jnp.int32),
        ),
    )(flat_e.reshape(1, TK))
    return pos.reshape(TK), emap.reshape(G), xmap.reshape(G), used.reshape(G)


@functools.lru_cache(maxsize=None)
def _build_sc_dispatch():
    mesh = plsc.VectorSubcoreMesh(core_axis_name="c", subcore_axis_name="s")

    @functools.partial(
        pl.kernel,
        mesh=mesh,
        out_type=jax.ShapeDtypeStruct((TKP, D), jnp.float32),
        scratch_types=[
            pltpu.VMEM((_TPW,), jnp.int32),
            pltpu.VMEM((_TPW,), jnp.int32),
            pltpu.VMEM((_TPW, D), jnp.float32),
            pltpu.SemaphoreType.DMA,
            pltpu.SemaphoreType.DMA,
        ],
        compiler_params=pltpu.CompilerParams(needs_layout_passes=False),
    )
    def sc_dispatch(x_hbm, pe_hbm, po_hbm, xp_hbm,
                    pos_a, pos_b, rows_a, sem_a, sem_b):
        wid = lax.axis_index("s") * _NC + lax.axis_index("c")
        t_base = wid * _TPW
        pltpu.sync_copy(pe_hbm.at[pl.ds(t_base, _TPW)], pos_a)
        pltpu.sync_copy(po_hbm.at[pl.ds(t_base, _TPW)], pos_b)
        pltpu.sync_copy(x_hbm.at[pl.ds(t_base, _TPW)], rows_a)
        sa = pltpu.async_copy(rows_a, xp_hbm.at[pos_a], sem_a)
        sb = pltpu.async_copy(rows_a, xp_hbm.at[pos_b], sem_b)
        sa.wait()
        sb.wait()

    return sc_dispatch


def _sc_dispatch(x, pos_e, pos_o):
    return _build_sc_dispatch()(x, pos_e, pos_o)


@functools.lru_cache(maxsize=None)
def _build_sc_combine():
    mesh = plsc.VectorSubcoreMesh(core_axis_name="c", subcore_axis_name="s")

    @functools.partial(
        pl.kernel,
        mesh=mesh,
        out_type=jax.ShapeDtypeStruct((T, D), jnp.float32),
        scratch_types=[
            pltpu.VMEM((_TPW,), jnp.int32),
            pltpu.VMEM((_TPW,), jnp.int32),
            pltpu.VMEM((_TPW, _LN), jnp.float32),
            pltpu.VMEM((_TPW, _LN), jnp.float32),
            pltpu.VMEM((_TPW, D), jnp.float32),
            pltpu.VMEM((_TPW, D), jnp.float32),
            pltpu.VMEM((_TPW // 4, D), jnp.float32),
            pltpu.SemaphoreType.DMA,
            pltpu.SemaphoreType.DMA,
        ],
        compiler_params=pltpu.CompilerParams(needs_layout_passes=False),
    )
    def sc_combine(outs_hbm, pos_hbm, wb_hbm, y_hbm,
                   pos_a, pos_b, wb_a, wb_b, rows_a, rows_b, y_v,
                   sem_a, sem_b):
        wid = lax.axis_index("s") * _NC + lax.axis_index("c")
        t_base = wid * _TPW
        yw = _TPW // 4
        pltpu.sync_copy(pos_hbm.at[pl.ds(t_base, _TPW)], pos_a)
        pltpu.sync_copy(pos_hbm.at[pl.ds(T + t_base, _TPW)], pos_b)
        pltpu.sync_copy(wb_hbm.at[pl.ds(t_base, _TPW)], wb_a)
        pltpu.sync_copy(wb_hbm.at[pl.ds(T + t_base, _TPW)], wb_b)
        ga = pltpu.async_copy(outs_hbm.at[pos_a], rows_a, sem_a)
        gb = pltpu.async_copy(outs_hbm.at[pos_b], rows_b, sem_b)
        ga.wait()
        gb.wait()

        for h in range(4):
            @plsc.parallel_loop(0, yw, unroll=4)
            def _wave(j, _h=h):
                jj = _h * yw + j
                w0 = wb_a[jj, :]
                w1 = wb_b[jj, :]
                for v in range(_VPR):
                    s = pl.ds(v * _LN, _LN)
                    y_v[j, s] = rows_a[jj, s] * w0 + rows_b[jj, s] * w1

            pltpu.sync_copy(y_v, y_hbm.at[pl.ds(t_base + h * yw, yw)])

    return sc_combine


def _sc_combine(outs, pos, wb):
    return _build_sc_combine()(outs, pos, wb)


def _ffn_body(em_ref, xm_ref, us_ref, x_ref, wg_ref, wu_ref, wd_ref, o_ref):
    g = pl.program_id(0)

    @pl.when(us_ref[g] == 1)
    def _compute():
        x = x_ref[...]
        gt = jnp.dot(x, wg_ref[0], preferred_element_type=jnp.float32)
        up = jnp.dot(x, wu_ref[0], preferred_element_type=jnp.float32)
        h = (gt * jax.nn.sigmoid(gt)) * up
        o_ref[...] = jnp.dot(h, wd_ref[0], preferred_element_type=jnp.float32)


def _grouped_ffn(emap, xmap, used, xp, Wg, Wu, Wd):
    grid_spec = pltpu.PrefetchScalarGridSpec(
        num_scalar_prefetch=3,
        grid=(G,),
        in_specs=[
            pl.BlockSpec((BLK, D), lambda g, em, xm, us: (xm[g], 0)),
            pl.BlockSpec((1, D, H), lambda g, em, xm, us: (em[g], 0, 0)),
            pl.BlockSpec((1, D, H), lambda g, em, xm, us: (em[g], 0, 0)),
            pl.BlockSpec((1, H, D), lambda g, em, xm, us: (em[g], 0, 0)),
        ],
        out_specs=pl.BlockSpec((BLK, D), lambda g, em, xm, us: (xm[g], 0)),
    )
    return pl.pallas_call(
        _ffn_body,
        grid_spec=grid_spec,
        out_shape=jax.ShapeDtypeStruct((TKP, D), jnp.float32),
    )(emap, xmap, used, xp, Wg, Wu, Wd)


def kernel(x, token_to_expert_indices, weights, Wg, Wu, Wd):
    idx = token_to_expert_indices.astype(jnp.int32)
    flat_e = idx.T.reshape(TK)
    wb = jnp.broadcast_to(weights.T.reshape(TK, 1), (TK, _LN))

    pos, emap, xmap, used = _metadata(flat_e)
    xp = _sc_dispatch(x, pos[:T], pos[T:])
    outs = _grouped_ffn(emap, xmap, used, xp, Wg, Wu, Wd)
    y = _sc_combine(outs, pos, wb)
    return y


# speedup vs baseline: 1.1053x; 1.1053x over previous
import functools

import jax
import jax.numpy as jnp
from jax import lax
from jax.experimental import pallas as pl
from jax.experimental.pallas import tpu as pltpu
from jax.experimental.pallas import tpu_sc as plsc

T, D, H, E, K = 2048, 768, 512, 16, 2
TK = T * K
BLK = 256
G = TK // BLK + E
TKP = G * BLK

_NC = 2
_NS = 16
_NW = _NC * _NS
_APW = TK // _NW
_HPW = _APW // 2
_TPW = T // _NW
_LN = 16
_VPR = D // _LN


def _meta_body(e_ref, pos_ref, emap_ref, xmap_ref, used_ref):
    ev = e_ref[...]
    lane_e = lax.broadcasted_iota(jnp.int32, (E, TK), 0)
    oneh = (jnp.broadcast_to(ev, (E, TK)) == lane_e).astype(jnp.int32)

    a = oneh
    k = 1
    while k < TK:
        a = a + jnp.concatenate(
            [jnp.zeros((E, k), jnp.int32), a[:, : TK - k]], axis=1)
        k *= 2
    counts = a[:, TK - 1:TK]
    rank = jnp.sum(oneh * a, axis=0, keepdims=True) - 1

    nblk = (counts + (BLK - 1)) // BLK
    b = nblk
    k = 1
    while k < E:
        b = b + jnp.concatenate(
            [jnp.zeros((k, 1), jnp.int32), b[: E - k, :]], axis=0)
        k *= 2
    blk_cum = b
    bstart = (blk_cum - nblk) * BLK
    pos_ref[...] = (
        jnp.sum(oneh * jnp.broadcast_to(bstart, (E, TK)), axis=0,
                keepdims=True) + rank)

    bid = lax.broadcasted_iota(jnp.int32, (1, G), 1)
    nused = jnp.sum(nblk)
    used = (bid < nused).astype(jnp.int32)
    eb = jnp.sum(
        (jnp.broadcast_to(bid, (E, G)) >= jnp.broadcast_to(blk_cum, (E, G)))
        .astype(jnp.int32), axis=0, keepdims=True)
    eb = jnp.minimum(eb, E - 1)
    last_used = jnp.maximum(nused - 1, 0)
    sel = (bid == last_used).astype(jnp.int32)
    e_last = jnp.sum(sel * eb)
    emap_ref[...] = jnp.where(used == 1, eb, e_last)
    xmap_ref[...] = jnp.minimum(bid, last_used)
    used_ref[...] = used


def _metadata(flat_e):
    pos, emap, xmap, used = pl.pallas_call(
        _meta_body,
        out_shape=(
            jax.ShapeDtypeStruct((1, TK), jnp.int32),
            jax.ShapeDtypeStruct((1, G), jnp.int32),
            jax.ShapeDtypeStruct((1, G), jnp.int32),
            jax.ShapeDtypeStruct((1, G), jnp.int32),
        ),
    )(flat_e.reshape(1, TK))
    return pos.reshape(TK), emap.reshape(G), xmap.reshape(G), used.reshape(G)


@functools.lru_cache(maxsize=None)
def _build_sc_dispatch():
    mesh = plsc.VectorSubcoreMesh(core_axis_name="c", subcore_axis_name="s")

    @functools.partial(
        pl.kernel,
        mesh=mesh,
        out_type=jax.ShapeDtypeStruct((TKP, D), jnp.float32),
        scratch_types=[
            pltpu.VMEM((_TPW,), jnp.int32),
            pltpu.VMEM((_TPW,), jnp.int32),
            pltpu.VMEM((_TPW, D), jnp.float32),
            pltpu.SemaphoreType.DMA,
            pltpu.SemaphoreType.DMA,
        ],
        compiler_params=pltpu.CompilerParams(needs_layout_passes=False),
    )
    def sc_dispatch(x_hbm, pe_hbm, po_hbm, xp_hbm,
                    pos_a, pos_b, rows_a, sem_a, sem_b):
        wid = lax.axis_index("s") * _NC + lax.axis_index("c")
        t_base = wid * _TPW
        pltpu.sync_copy(pe_hbm.at[pl.ds(t_base, _TPW)], pos_a)
        pltpu.sync_copy(po_hbm.at[pl.ds(t_base, _TPW)], pos_b)
        pltpu.sync_copy(x_hbm.at[pl.ds(t_base, _TPW)], rows_a)
        sa = pltpu.async_copy(rows_a, xp_hbm.at[pos_a], sem_a)
        sb = pltpu.async_copy(rows_a, xp_hbm.at[pos_b], sem_b)
        sa.wait()
        sb.wait()

    return sc_dispatch


def _sc_dispatch(x, pos_e, pos_o):
    return _build_sc_dispatch()(x, pos_e, pos_o)


@functools.lru_cache(maxsize=None)
def _build_sc_combine():
    mesh = plsc.VectorSubcoreMesh(core_axis_name="c", subcore_axis_name="s")

    @functools.partial(
        pl.kernel,
        mesh=mesh,
        out_type=jax.ShapeDtypeStruct((T, D), jnp.float32),
        scratch_types=[
            pltpu.VMEM((_TPW,), jnp.int32),
            pltpu.VMEM((_TPW,), jnp.int32),
            pltpu.VMEM((_TPW, _LN), jnp.float32),
            pltpu.VMEM((_TPW, _LN), jnp.float32),
            pltpu.VMEM((_TPW, D), jnp.float32),
            pltpu.VMEM((_TPW, D), jnp.float32),
            pltpu.VMEM((_TPW // 4, D), jnp.float32),
            pltpu.SemaphoreType.DMA,
            pltpu.SemaphoreType.DMA,
        ],
        compiler_params=pltpu.CompilerParams(needs_layout_passes=False),
    )
    def sc_combine(outs_hbm, pos_hbm, wb_hbm, y_hbm,
                   pos_a, pos_b, wb_a, wb_b, rows_a, rows_b, y_v,
                   sem_a, sem_b):
        wid = lax.axis_index("s") * _NC + lax.axis_index("c")
        t_base = wid * _TPW
        yw = _TPW // 4
        pltpu.sync_copy(pos_hbm.at[pl.ds(t_base, _TPW)], pos_a)
        pltpu.sync_copy(pos_hbm.at[pl.ds(T + t_base, _TPW)], pos_b)
        pltpu.sync_copy(wb_hbm.at[pl.ds(t_base, _TPW)], wb_a)
        pltpu.sync_copy(wb_hbm.at[pl.ds(T + t_base, _TPW)], wb_b)
        ga = pltpu.async_copy(outs_hbm.at[pos_a], rows_a, sem_a)
        gb = pltpu.async_copy(outs_hbm.at[pos_b], rows_b, sem_b)
        ga.wait()
        gb.wait()

        for h in range(4):
            @plsc.parallel_loop(0, yw, unroll=4)
            def _wave(j, _h=h):
                jj = _h * yw + j
                w0 = wb_a[jj, :]
                w1 = wb_b[jj, :]
                for v in range(_VPR):
                    s = pl.ds(v * _LN, _LN)
                    y_v[j, s] = rows_a[jj, s] * w0 + rows_b[jj, s] * w1

            pltpu.sync_copy(y_v, y_hbm.at[pl.ds(t_base + h * yw, yw)])

    return sc_combine


def _sc_combine(outs, pos, wb):
    return _build_sc_combine()(outs, pos, wb)


def _ffn_body(em_ref, xm_ref, us_ref, x_ref, wg_ref, wu_ref, wd_ref, o_ref):
    g = pl.program_id(0)

    @pl.when(us_ref[g] == 1)
    def _compute():
        x = x_ref[...]
        gt = jnp.dot(x, wg_ref[0], preferred_element_type=jnp.float32)
        up = jnp.dot(x, wu_ref[0], preferred_element_type=jnp.float32)
        h = (gt * jax.nn.sigmoid(gt)) * up
        o_ref[...] = jnp.dot(h, wd_ref[0], preferred_element_type=jnp.float32)


def _grouped_ffn(emap, xmap, used, xp, Wg, Wu, Wd):
    grid_spec = pltpu.PrefetchScalarGridSpec(
        num_scalar_prefetch=3,
        grid=(G,),
        in_specs=[
            pl.BlockSpec((BLK, D), lambda g, em, xm, us: (xm[g], 0)),
            pl.BlockSpec((1, D, H), lambda g, em, xm, us: (em[g], 0, 0)),
            pl.BlockSpec((1, D, H), lambda g, em, xm, us: (em[g], 0, 0)),
            pl.BlockSpec((1, H, D), lambda g, em, xm, us: (em[g], 0, 0)),
        ],
        out_specs=pl.BlockSpec((BLK, D), lambda g, em, xm, us: (xm[g], 0)),
    )
    return pl.pallas_call(
        _ffn_body,
        grid_spec=grid_spec,
        out_shape=jax.ShapeDtypeStruct((TKP, D), jnp.float32),
    )(emap, xmap, used, xp, Wg, Wu, Wd)


def kernel(x, token_to_expert_indices, weights, Wg, Wu, Wd):
    idx = token_to_expert_indices.astype(jnp.int32)
    flat_e = idx.T.reshape(TK)
    wb = jnp.broadcast_to(weights.T.reshape(TK, 1), (TK, _LN))

    pos, emap, xmap, used = _metadata(flat_e)
    xp = _sc_dispatch(x, pos[:T], pos[T:])
    outs = _grouped_ffn(emap, xmap, used, xp, Wg, Wu, Wd)
    y = _sc_combine(outs, pos, wb)
    return y
